# baseline (speedup 1.0000x reference)
import jax
import jax.numpy as jnp
from jax import lax
from jax.experimental import pallas as pl
from jax.experimental.pallas import tpu as pltpu
from jax.experimental.pallas import tpu_sc as plsc

VOCAB = 50272
D = 768
MAX_POS = 2048
OFFSET = 2
B, S = 4, 2048

NC, NS = 2, 16
NW = NC * NS
LANES = 16
SPW = S // NW
CH = 16
NCHUNK = B * SPW // CH
NBUF = 6
LOOK = 4
QPB = SPW // CH
CCHUNKS = D // LANES


def _emb_body(ids_hbm, tok_hbm, pos_hbm, out_hbm,
              idx_all, pos_idx, pos_v, *bufs_and_sems):
    bufs = bufs_and_sems[:NBUF]
    sem_p, sem_i = bufs_and_sems[NBUF:NBUF + 2]
    sem_g = bufs_and_sems[NBUF + 2:NBUF + 2 + NBUF]
    sem_o = bufs_and_sems[NBUF + 2 + NBUF:]

    wid = lax.axis_index("s") * NC + lax.axis_index("c")
    s0 = wid * SPW

    id_cps = [
        pltpu.async_copy(ids_hbm.at[pl.ds(b * S + s0, SPW)],
                         idx_all.at[pl.ds(b * SPW, SPW)], sem_i)
        for b in range(B)
    ]

    for c in range(SPW // LANES):
        pos_idx[pl.ds(c * LANES, LANES)] = (
            lax.iota(jnp.int32, LANES) + (OFFSET + s0 + c * LANES)
        )
    pos_cp = pltpu.async_copy(pos_hbm.at[pos_idx], pos_v, sem_p)

    def g_start(k):
        m = k % NBUF
        pltpu.async_copy(tok_hbm.at[idx_all.at[pl.ds(k * CH, CH)]],
                         bufs[m], sem_g[m])

    def g_wait(k):
        m = k % NBUF
        pltpu.make_async_copy(tok_hbm.at[idx_all.at[pl.ds(0, CH)]],
                              bufs[m], sem_g[m]).wait()

    def o_start(k):
        m = k % NBUF
        pltpu.async_copy(bufs[m],
                         out_hbm.at[k // QPB, pl.ds(s0 + (k % QPB) * CH, CH)],
                         sem_o[m])

    def o_wait(k):
        m = k % NBUF
        pltpu.make_async_copy(bufs[m], out_hbm.at[0, pl.ds(s0, CH)],
                              sem_o[m]).wait()

    for cp in id_cps:
        cp.wait()
    for k in range(LOOK):
        g_start(k)
    pos_cp.wait()

    for k in range(NCHUNK):
        g_wait(k)
        if k + LOOK < NCHUNK:
            if k + LOOK - NBUF >= 0:
                o_wait(k + LOOK - NBUF)
            g_start(k + LOOK)

        buf = bufs[k % NBUF]
        poff = (k % QPB) * CH

        @plsc.parallel_loop(0, CH)
        def _add_row(r):
            for c in range(CCHUNKS):
                sl = pl.ds(c * LANES, LANES)
                plsc.addupdate(buf.at[r, sl], pos_v[poff + r, sl])

        o_start(k)

    for k in range(NCHUNK - NBUF, NCHUNK):
        o_wait(k)


@jax.jit
def _emb(ids_flat, tok_table, pos_table):
    mesh = plsc.VectorSubcoreMesh(core_axis_name="c", subcore_axis_name="s")
    return pl.kernel(
        _emb_body,
        out_type=jax.ShapeDtypeStruct((B, S, D), jnp.float32),
        mesh=mesh,
        scratch_types=[
            pltpu.VMEM((B * SPW,), jnp.int32),
            pltpu.VMEM((SPW,), jnp.int32),
            pltpu.VMEM((SPW, D), jnp.float32),
        ] + [pltpu.VMEM((CH, D), jnp.float32)] * NBUF
          + [pltpu.SemaphoreType.DMA] * (2 + 2 * NBUF),
    )(ids_flat, tok_table, pos_table)


def kernel(input_ids, tok_table, pos_table):
    ids_flat = input_ids.astype(jnp.int32).reshape(B * S)
    return _emb(ids_flat, tok_table, pos_table)
